# Initial kernel scaffold; baseline (speedup 1.0000x reference)
import jax
import jax.numpy as jnp
from jax.experimental import pallas as pl


def kernel(x, edge_index, W_in, b_in, gammas, betas, Wc, bc, W_out, b_out):
    raise NotImplementedError("write your pallas kernel here")


# speedup vs baseline: 9.9119x; 9.9119x over previous
import functools

import jax
import jax.numpy as jnp
from jax import lax
from jax.experimental import pallas as pl
from jax.experimental.pallas import tpu as pltpu
from jax.experimental.pallas import tpu_sc as plsc

D = 128
EPS = 1e-5
NC = 2
NS = 16
NW = NC * NS
CH = 128
N_PAD = 10240
ROWS_PER_TILE = N_PAD // NS
CPB = 128
NB = ROWS_PER_TILE // CPB


def _sc_mesh():
    return plsc.VectorSubcoreMesh(core_axis_name="c", subcore_axis_name="s")


@functools.partial(
    pl.kernel,
    out_type=jax.ShapeDtypeStruct((NC, N_PAD, D), jnp.float32),
    mesh=_sc_mesh(),
    scratch_types=[
        pltpu.VMEM((CH,), jnp.int32),
        pltpu.VMEM((CH, D), jnp.float32),
        pltpu.VMEM((CPB, D), jnp.float32),
        pltpu.VMEM_SHARED((N_PAD, D), jnp.float32),
    ],
)
def _deg_kernel(dst_hbm, out_hbm, didx, ones_v, stage, acc):
    c = lax.axis_index("c")
    s = lax.axis_index("s")
    wid = c * NS + s
    ept = dst_hbm.shape[0] // NW
    nchunk = ept // CH

    def _fill(i, _):
        for j in range(D // 16):
            ones_v[i, pl.ds(j * 16, 16)] = jnp.ones((16,), jnp.float32)
            stage[i, pl.ds(j * 16, 16)] = jnp.zeros((16,), jnp.float32)
        return 0

    lax.fori_loop(0, CH, _fill, 0)
    for b in range(NB):
        pltpu.sync_copy(stage, acc.at[pl.ds(s * ROWS_PER_TILE + b * CPB, CPB)])
    plsc.subcore_barrier()

    def _body(j, _):
        base = wid * ept + j * CH
        pltpu.sync_copy(dst_hbm.at[pl.ds(base, CH)], didx)
        pltpu.sync_copy(ones_v, acc.at[didx], add=True)
        return 0

    lax.fori_loop(0, nchunk, _body, 0)
    plsc.subcore_barrier()
    for b in range(NB):
        r0 = s * ROWS_PER_TILE + b * CPB
        pltpu.sync_copy(acc.at[pl.ds(r0, CPB)], stage)
        pltpu.sync_copy(stage, out_hbm.at[c, pl.ds(r0, CPB)])


@functools.partial(
    pl.kernel,
    out_type=jax.ShapeDtypeStruct((NC, N_PAD, D), jnp.float32),
    mesh=_sc_mesh(),
    scratch_types=[
        pltpu.VMEM((CH,), jnp.int32),
        pltpu.VMEM((CH,), jnp.int32),
        pltpu.VMEM((CH, D), jnp.float32),
        pltpu.SemaphoreType.DMA,
        pltpu.VMEM_SHARED((N_PAD, D), jnp.float32),
    ],
)
def _agg_kernel(table_hbm, src_hbm, dst_hbm, out_hbm, sidx, didx, rows, sem, acc):
    c = lax.axis_index("c")
    s = lax.axis_index("s")
    wid = c * NS + s
    ept = src_hbm.shape[0] // NW
    nchunk = ept // CH

    def _zero(i, _):
        for j in range(D // 16):
            rows[i, pl.ds(j * 16, 16)] = jnp.zeros((16,), jnp.float32)
        return 0

    lax.fori_loop(0, CH, _zero, 0)
    for b in range(NB):
        pltpu.sync_copy(rows, acc.at[pl.ds(s * ROWS_PER_TILE + b * CPB, CPB)])
    plsc.subcore_barrier()

    def _body(j, _):
        base = wid * ept + j * CH
        pltpu.sync_copy(src_hbm.at[pl.ds(base, CH)], sidx)
        pltpu.sync_copy(dst_hbm.at[pl.ds(base, CH)], didx)
        pltpu.async_copy(table_hbm.at[sidx], rows, sem).wait()
        pltpu.sync_copy(rows, acc.at[didx], add=True)
        return 0

    lax.fori_loop(0, nchunk, _body, 0)
    plsc.subcore_barrier()
    for b in range(NB):
        r0 = s * ROWS_PER_TILE + b * CPB
        pltpu.sync_copy(acc.at[pl.ds(r0, CPB)], rows)
        pltpu.sync_copy(rows, out_hbm.at[c, pl.ds(r0, CPB)])


def _k1_body(x_ref, w_ref, b_ref, h_ref, st_ref):
    i = pl.program_id(0)
    h = jnp.maximum(
        jnp.dot(x_ref[...], w_ref[...], preferred_element_type=jnp.float32)
        + b_ref[...], 0.0)
    h_ref[...] = h
    s1 = jnp.sum(h, axis=0, keepdims=True)
    s2 = jnp.sum(h * h, axis=0, keepdims=True)
    blk = jnp.concatenate([s1, s2, jnp.zeros((6, D), jnp.float32)], axis=0)

    @pl.when(i == 0)
    def _():
        st_ref[...] = blk

    @pl.when(i != 0)
    def _():
        st_ref[...] += blk


def _dinv(d0_ref, d1_ref):
    return lax.rsqrt(d0_ref[0, :, 0:1] + d1_ref[0, :, 0:1] + 1.0)


def _bn(h, st_ref, g_ref, bt_ref, n):
    mean = st_ref[0:1, :] / n
    var = st_ref[1:2, :] / n - mean * mean
    rstd = lax.rsqrt(var + EPS)
    return (h - mean) * (rstd * g_ref[...]) + bt_ref[...]


def _k2_body(h_ref, st_ref, d0_ref, d1_ref, g_ref, bt_ref, w_ref, o_ref, *, n):
    hn = _bn(h_ref[...], st_ref, g_ref, bt_ref, n)
    hw = jnp.dot(hn, w_ref[...], preferred_element_type=jnp.float32)
    o_ref[...] = hw * _dinv(d0_ref, d1_ref)


def _k3_body(q0_ref, q1_ref, hws_ref, d0_ref, d1_ref, bc_ref, h_ref, st_ref):
    i = pl.program_id(0)
    agg = (q0_ref[0] + q1_ref[0] + hws_ref[...]) * _dinv(d0_ref, d1_ref) \
        + bc_ref[...]
    h = jnp.maximum(agg, 0.0)
    h_ref[...] = h
    s1 = jnp.sum(h, axis=0, keepdims=True)
    s2 = jnp.sum(h * h, axis=0, keepdims=True)
    blk = jnp.concatenate([s1, s2, jnp.zeros((6, D), jnp.float32)], axis=0)

    @pl.when(i == 0)
    def _():
        st_ref[...] = blk

    @pl.when(i != 0)
    def _():
        st_ref[...] += blk


def _k5_body(q0_ref, q1_ref, hws_ref, d0_ref, d1_ref, bc_ref, w_ref, b_ref,
             o_ref):
    agg = (q0_ref[0] + q1_ref[0] + hws_ref[...]) * _dinv(d0_ref, d1_ref) \
        + bc_ref[...]
    h = jnp.maximum(agg, 0.0)
    o_ref[...] = (
        jnp.dot(h, w_ref[...], preferred_element_type=jnp.float32)
        + b_ref[...])


def kernel(x, edge_index, W_in, b_in, gammas, betas, Wc, bc, W_out, b_out):
    n, d_in = x.shape
    e = edge_index.shape[1]
    d_out = W_out.shape[1]
    src = edge_index[0].astype(jnp.int32)
    dst = edge_index[1].astype(jnp.int32)
    ept = -(-e // (NW * CH)) * CH
    pad = ept * NW - e
    src_p = jnp.concatenate([src, jnp.zeros((pad,), jnp.int32)])
    dst_p = jnp.concatenate([dst, jnp.full((pad,), N_PAD - 1, jnp.int32)])

    R = 1000
    G = n // R
    fnn = float(n)

    row_spec = pl.BlockSpec((R, D), lambda i: (i, 0))

    def cspec(shape):
        return pl.BlockSpec(shape, lambda i, _s=len(shape): (0,) * _s)

    deg0_spec = pl.BlockSpec((1, R, D), lambda i: (0, i, 0))
    deg1_spec = pl.BlockSpec((1, R, D), lambda i: (1, i, 0))
    q0_spec = pl.BlockSpec((1, R, D), lambda i: (0, i, 0))
    q1_spec = pl.BlockSpec((1, R, D), lambda i: (1, i, 0))

    h0, st0 = pl.pallas_call(
        _k1_body,
        grid=(G,),
        in_specs=[pl.BlockSpec((R, d_in), lambda i: (i, 0)),
                  cspec((d_in, D)), cspec((1, D))],
        out_specs=[row_spec, cspec((8, D))],
        out_shape=[jax.ShapeDtypeStruct((n, D), jnp.float32),
                   jax.ShapeDtypeStruct((8, D), jnp.float32)],
    )(x, W_in, b_in[None, :])

    degp = _deg_kernel(dst_p)

    def apply_bn(h, st, gamma, beta, W):
        return pl.pallas_call(
            functools.partial(_k2_body, n=fnn),
            grid=(G,),
            in_specs=[row_spec, cspec((8, D)), deg0_spec, deg1_spec,
                      cspec((1, D)), cspec((1, D)), cspec((D, D))],
            out_specs=row_spec,
            out_shape=jax.ShapeDtypeStruct((n, D), jnp.float32),
        )(h, st, degp, degp, gamma[None, :], beta[None, :], W)

    hws0 = apply_bn(h0, st0, gammas[0], betas[0], Wc[0])
    q = _agg_kernel(hws0, src_p, dst_p)
    h1, st1 = pl.pallas_call(
        _k3_body,
        grid=(G,),
        in_specs=[q0_spec, q1_spec, row_spec, deg0_spec, deg1_spec,
                  cspec((1, D))],
        out_specs=[row_spec, cspec((8, D))],
        out_shape=[jax.ShapeDtypeStruct((n, D), jnp.float32),
                   jax.ShapeDtypeStruct((8, D), jnp.float32)],
    )(q, q, hws0, degp, degp, bc[0][None, :])

    hws1 = apply_bn(h1, st1, gammas[1], betas[1], Wc[1])
    q2 = _agg_kernel(hws1, src_p, dst_p)
    out = pl.pallas_call(
        _k5_body,
        grid=(G,),
        in_specs=[q0_spec, q1_spec, row_spec, deg0_spec, deg1_spec,
                  cspec((1, D)), cspec((D, d_out)), cspec((1, d_out))],
        out_specs=pl.BlockSpec((R, d_out), lambda i: (i, 0)),
        out_shape=jax.ShapeDtypeStruct((n, d_out), jnp.float32),
    )(q2, q2, hws1, degp, degp, bc[1][None, :], W_out, b_out[None, :])
    return out


# speedup vs baseline: 10.3214x; 1.0413x over previous
import functools

import jax
import jax.numpy as jnp
from jax import lax
from jax.experimental import pallas as pl
from jax.experimental.pallas import tpu as pltpu
from jax.experimental.pallas import tpu_sc as plsc

D = 128
EPS = 1e-5
NC = 2
NS = 16
NW = NC * NS
CH = 128
NCH = 80
RB = 2
SK = 8
NST = NCH // SK
N_PAD = 10112
ROWS_PER_TILE = N_PAD // NS
ZBLOCKS = [(0, 128), (128, 128), (256, 128), (384, 128), (512, 120)]


def _sc_mesh():
    return plsc.VectorSubcoreMesh(core_axis_name="c", subcore_axis_name="s")


@functools.partial(
    pl.kernel,
    out_type=jax.ShapeDtypeStruct((NC, N_PAD, D), jnp.float32),
    mesh=_sc_mesh(),
    scratch_types=[
        pltpu.VMEM((NCH, CH), jnp.int32),
        pltpu.VMEM((CH, D), jnp.float32),
        pltpu.VMEM((CH, D), jnp.float32),
        pltpu.SemaphoreType.DMA,
        pltpu.VMEM_SHARED((N_PAD, D), jnp.float32),
    ],
)
def _deg_kernel(dst_hbm, out_hbm, didx2, ones_v, stage, sem, acc):
    c = lax.axis_index("c")
    s = lax.axis_index("s")
    wid = c * NS + s
    pltpu.sync_copy(dst_hbm.at[wid], didx2)

    def _fill(i, _):
        for j in range(D // 16):
            ones_v[i, pl.ds(j * 16, 16)] = jnp.ones((16,), jnp.float32)
            stage[i, pl.ds(j * 16, 16)] = jnp.zeros((16,), jnp.float32)
        return 0

    lax.fori_loop(0, CH, _fill, 0)
    r0 = s * ROWS_PER_TILE
    for off, sz in ZBLOCKS:
        pltpu.sync_copy(stage.at[pl.ds(0, sz)], acc.at[pl.ds(r0 + off, sz)])
    plsc.subcore_barrier()

    for j in range(RB):
        pltpu.async_copy(ones_v, acc.at[didx2.at[j]], sem, add=True)

    def _body(j, _):
        d = pltpu.async_copy(ones_v, acc.at[didx2.at[j]], sem, add=True)
        d.wait()
        return 0

    lax.fori_loop(RB, NCH, _body, 0)
    for j in range(RB):
        pltpu.make_async_copy(ones_v, acc.at[didx2.at[j]], sem).wait()
    plsc.subcore_barrier()
    for off, sz in ZBLOCKS:
        pltpu.sync_copy(acc.at[pl.ds(r0 + off, sz)], stage.at[pl.ds(0, sz)])
        pltpu.sync_copy(stage.at[pl.ds(0, sz)],
                        out_hbm.at[c, pl.ds(r0 + off, sz)])


@functools.partial(
    pl.kernel,
    out_type=jax.ShapeDtypeStruct((NC, N_PAD, D), jnp.float32),
    mesh=_sc_mesh(),
    scratch_types=[
        pltpu.VMEM((2, SK, CH), jnp.int32),
        pltpu.VMEM((2, SK, CH), jnp.int32),
        pltpu.VMEM((RB, CH, D), jnp.float32),
        pltpu.SemaphoreType.DMA,
        pltpu.SemaphoreType.DMA,
        pltpu.SemaphoreType.DMA,
        pltpu.VMEM_SHARED((N_PAD, D), jnp.float32),
    ],
)
def _agg_kernel(table_hbm, src_hbm, dst_hbm, out_hbm, sidx, didx, ring,
                gsem, ssem, isem, acc):
    c = lax.axis_index("c")
    s = lax.axis_index("s")
    wid = c * NS + s

    pltpu.sync_copy(src_hbm.at[wid, pl.ds(0, SK)], sidx.at[0])
    pltpu.sync_copy(dst_hbm.at[wid, pl.ds(0, SK)], didx.at[0])
    pltpu.async_copy(src_hbm.at[wid, pl.ds(SK, SK)], sidx.at[1], isem)
    pltpu.async_copy(dst_hbm.at[wid, pl.ds(SK, SK)], didx.at[1], isem)

    zslot = ring.at[0]

    def _zero(i, _):
        for j in range(D // 16):
            zslot[i, pl.ds(j * 16, 16)] = jnp.zeros((16,), jnp.float32)
        return 0

    lax.fori_loop(0, CH, _zero, 0)
    r0 = s * ROWS_PER_TILE
    for off, sz in ZBLOCKS:
        pltpu.sync_copy(ring.at[0, pl.ds(0, sz)],
                        acc.at[pl.ds(r0 + off, sz)])
    plsc.subcore_barrier()

    for b in range(RB):
        pltpu.async_copy(table_hbm.at[sidx.at[0, b]], ring.at[b], gsem)

    def _stage(t, _):
        st = t % 2
        nx = 1 - st
        for b in range(SK):
            pltpu.make_async_copy(
                table_hbm.at[sidx.at[st, b]], ring.at[b % RB], gsem).wait()
            pltpu.async_copy(ring.at[b % RB], acc.at[didx.at[st, b]], ssem,
                             add=True).wait()
            if b == SK - 2:
                @pl.when(t + 1 < NST)
                def _():
                    pltpu.make_async_copy(
                        src_hbm.at[wid, pl.ds(0, SK)], sidx.at[nx],
                        isem).wait()
                    pltpu.make_async_copy(
                        dst_hbm.at[wid, pl.ds(0, SK)], didx.at[nx],
                        isem).wait()
                    pltpu.async_copy(table_hbm.at[sidx.at[nx, 0]],
                                     ring.at[b % RB], gsem)
            elif b == SK - 1:
                @pl.when(t + 2 < NST)
                def _():
                    off = (t + 2) * SK
                    pltpu.async_copy(src_hbm.at[wid, pl.ds(off, SK)],
                                     sidx.at[st], isem)
                    pltpu.async_copy(dst_hbm.at[wid, pl.ds(off, SK)],
                                     didx.at[st], isem)

                @pl.when(t + 1 < NST)
                def _():
                    pltpu.async_copy(table_hbm.at[sidx.at[nx, 1]],
                                     ring.at[b % RB], gsem)
            else:
                pltpu.async_copy(table_hbm.at[sidx.at[st, b + RB]],
                                 ring.at[b % RB], gsem)
        return 0

    lax.fori_loop(0, NST, _stage, 0)
    plsc.subcore_barrier()
    for off, sz in ZBLOCKS:
        pltpu.sync_copy(acc.at[pl.ds(r0 + off, sz)],
                        ring.at[0, pl.ds(0, sz)])
        pltpu.sync_copy(ring.at[0, pl.ds(0, sz)],
                        out_hbm.at[c, pl.ds(r0 + off, sz)])


def _k1_body(x_ref, w_ref, b_ref, h_ref, st_ref):
    i = pl.program_id(0)
    h = jnp.maximum(
        jnp.dot(x_ref[...], w_ref[...], preferred_element_type=jnp.float32)
        + b_ref[...], 0.0)
    h_ref[...] = h
    s1 = jnp.sum(h, axis=0, keepdims=True)
    s2 = jnp.sum(h * h, axis=0, keepdims=True)
    blk = jnp.concatenate([s1, s2, jnp.zeros((6, D), jnp.float32)], axis=0)

    @pl.when(i == 0)
    def _():
        st_ref[...] = blk

    @pl.when(i != 0)
    def _():
        st_ref[...] += blk


def _dinv(d0_ref, d1_ref):
    return lax.rsqrt(d0_ref[0, :, 0:1] + d1_ref[0, :, 0:1] + 1.0)


def _bn(h, st_ref, g_ref, bt_ref, n):
    mean = st_ref[0:1, :] / n
    var = st_ref[1:2, :] / n - mean * mean
    rstd = lax.rsqrt(var + EPS)
    return (h - mean) * (rstd * g_ref[...]) + bt_ref[...]


def _k2_body(h_ref, st_ref, d0_ref, d1_ref, g_ref, bt_ref, w_ref, o_ref, *, n):
    hn = _bn(h_ref[...], st_ref, g_ref, bt_ref, n)
    hw = jnp.dot(hn, w_ref[...], preferred_element_type=jnp.float32)
    o_ref[...] = hw * _dinv(d0_ref, d1_ref)


def _k3_body(q0_ref, q1_ref, hws_ref, d0_ref, d1_ref, bc_ref, h_ref, st_ref):
    i = pl.program_id(0)
    agg = (q0_ref[0] + q1_ref[0] + hws_ref[...]) * _dinv(d0_ref, d1_ref) \
        + bc_ref[...]
    h = jnp.maximum(agg, 0.0)
    h_ref[...] = h
    s1 = jnp.sum(h, axis=0, keepdims=True)
    s2 = jnp.sum(h * h, axis=0, keepdims=True)
    blk = jnp.concatenate([s1, s2, jnp.zeros((6, D), jnp.float32)], axis=0)

    @pl.when(i == 0)
    def _():
        st_ref[...] = blk

    @pl.when(i != 0)
    def _():
        st_ref[...] += blk


def _k5_body(q0_ref, q1_ref, hws_ref, d0_ref, d1_ref, bc_ref, w_ref, b_ref,
             o_ref):
    agg = (q0_ref[0] + q1_ref[0] + hws_ref[...]) * _dinv(d0_ref, d1_ref) \
        + bc_ref[...]
    h = jnp.maximum(agg, 0.0)
    o_ref[...] = (
        jnp.dot(h, w_ref[...], preferred_element_type=jnp.float32)
        + b_ref[...])


def kernel(x, edge_index, W_in, b_in, gammas, betas, Wc, bc, W_out, b_out):
    n, d_in = x.shape
    e = edge_index.shape[1]
    d_out = W_out.shape[1]
    src = edge_index[0].astype(jnp.int32)
    dst = edge_index[1].astype(jnp.int32)
    pad = NW * NCH * CH - e
    src_p = jnp.concatenate(
        [src, jnp.zeros((pad,), jnp.int32)]).reshape(NW, NCH, CH)
    dst_p = jnp.concatenate(
        [dst, jnp.full((pad,), N_PAD - 1, jnp.int32)]).reshape(NW, NCH, CH)

    R = 1000
    G = n // R
    fnn = float(n)

    row_spec = pl.BlockSpec((R, D), lambda i: (i, 0))

    def cspec(shape):
        return pl.BlockSpec(shape, lambda i, _s=len(shape): (0,) * _s)

    deg0_spec = pl.BlockSpec((1, R, D), lambda i: (0, i, 0))
    deg1_spec = pl.BlockSpec((1, R, D), lambda i: (1, i, 0))
    q0_spec = pl.BlockSpec((1, R, D), lambda i: (0, i, 0))
    q1_spec = pl.BlockSpec((1, R, D), lambda i: (1, i, 0))

    h0, st0 = pl.pallas_call(
        _k1_body,
        grid=(G,),
        in_specs=[pl.BlockSpec((R, d_in), lambda i: (i, 0)),
                  cspec((d_in, D)), cspec((1, D))],
        out_specs=[row_spec, cspec((8, D))],
        out_shape=[jax.ShapeDtypeStruct((n, D), jnp.float32),
                   jax.ShapeDtypeStruct((8, D), jnp.float32)],
    )(x, W_in, b_in[None, :])

    degp = _deg_kernel(dst_p)

    def apply_bn(h, st, gamma, beta, W):
        return pl.pallas_call(
            functools.partial(_k2_body, n=fnn),
            grid=(G,),
            in_specs=[row_spec, cspec((8, D)), deg0_spec, deg1_spec,
                      cspec((1, D)), cspec((1, D)), cspec((D, D))],
            out_specs=row_spec,
            out_shape=jax.ShapeDtypeStruct((n, D), jnp.float32),
        )(h, st, degp, degp, gamma[None, :], beta[None, :], W)

    hws0 = apply_bn(h0, st0, gammas[0], betas[0], Wc[0])
    q = _agg_kernel(hws0, src_p, dst_p)
    h1, st1 = pl.pallas_call(
        _k3_body,
        grid=(G,),
        in_specs=[q0_spec, q1_spec, row_spec, deg0_spec, deg1_spec,
                  cspec((1, D))],
        out_specs=[row_spec, cspec((8, D))],
        out_shape=[jax.ShapeDtypeStruct((n, D), jnp.float32),
                   jax.ShapeDtypeStruct((8, D), jnp.float32)],
    )(q, q, hws0, degp, degp, bc[0][None, :])

    hws1 = apply_bn(h1, st1, gammas[1], betas[1], Wc[1])
    q2 = _agg_kernel(hws1, src_p, dst_p)
    out = pl.pallas_call(
        _k5_body,
        grid=(G,),
        in_specs=[q0_spec, q1_spec, row_spec, deg0_spec, deg1_spec,
                  cspec((1, D)), cspec((D, d_out)), cspec((1, d_out))],
        out_specs=pl.BlockSpec((R, d_out), lambda i: (i, 0)),
        out_shape=jax.ShapeDtypeStruct((n, d_out), jnp.float32),
    )(q2, q2, hws1, degp, degp, bc[1][None, :], W_out, b_out[None, :])
    return out
